# Initial kernel scaffold; baseline (speedup 1.0000x reference)
import jax
import jax.numpy as jnp
from jax.experimental import pallas as pl


def kernel(nc_x, nc_edge_index, nc_edge_weight, dis_x, dis_edge_index, dis_edge_weight, NW1, Nb1, NW2, Nb2, DW1, Db1, DW2, Db2):
    raise NotImplementedError("write your pallas kernel here")


# speedup vs baseline: 4.4373x; 4.4373x over previous
import jax
import jax.numpy as jnp
from jax import lax
from jax.experimental import pallas as pl
from jax.experimental.pallas import tpu as pltpu
from jax.experimental.pallas import tpu_sc as plsc

N = 10000
NP = 10240
E = 160000
NS = 16
ET = E // NS
CH = 80
NCH = ET // CH
NR = NP // NS
ZR = 64
BLK = 1000
BLKP = 1024

_sc_mesh = plsc.VectorSubcoreMesh(
    core_axis_name="c", subcore_axis_name="s", num_cores=2, num_subcores=16)


def _deg_body(e_hbm, w_hbm, deg_hbm, eb, wb, zb, shared):
    c = lax.axis_index("c")
    s = lax.axis_index("s")

    def _z(i, _):
        zb[pl.ds(i * 16, 16)] = jnp.zeros((16,), jnp.float32)
        return 0
    lax.fori_loop(0, NR // 16, _z, 0)
    pltpu.sync_copy(zb, shared.at[pl.ds(s * NR, NR)])
    plsc.subcore_barrier()

    def _acc(ci, _):
        pltpu.sync_copy(e_hbm.at[c, s, ci], eb)
        pltpu.sync_copy(w_hbm.at[c, s, ci], wb)
        pltpu.sync_copy(wb, shared.at[eb.at[1]], add=True)
        return 0
    lax.fori_loop(0, NCH, _acc, 0)
    plsc.subcore_barrier()

    pltpu.sync_copy(shared.at[pl.ds(s * NR, NR)], zb)
    pltpu.sync_copy(zb, deg_hbm.at[c, pl.ds(s * NR, NR)])


_deg_call = pl.kernel(
    _deg_body,
    out_type=jax.ShapeDtypeStruct((2, NP), jnp.float32),
    mesh=_sc_mesh,
    scratch_types=[
        pltpu.VMEM((2, CH), jnp.int32),
        pltpu.VMEM((CH,), jnp.float32),
        pltpu.VMEM((NR,), jnp.float32),
        pltpu.VMEM_SHARED((NP,), jnp.float32),
    ],
)


def _conv_body(h1s_hbm, e_hbm, w_hbm, dinv_hbm, b1_hbm, u_hbm, v_hbm,
               eb, wb, rows, zb, hrows, dinv_v, b1_v, sem, shared):
    c = lax.axis_index("c")
    s = lax.axis_index("s")
    pltpu.sync_copy(dinv_hbm.at[c, pl.ds(s * NR, NR)], dinv_v)
    pltpu.sync_copy(b1_hbm.at[c], b1_v)

    def _sweep(gather_ref):
        def _z(i, _):
            for k in range(8):
                zb[i, pl.ds(k * 16, 16)] = jnp.zeros((16,), jnp.float32)
            return 0
        lax.fori_loop(0, ZR, _z, 0)

        def _zc(t, _):
            pltpu.sync_copy(zb, shared.at[pl.ds(s * NR + t * ZR, ZR)])
            return 0
        lax.fori_loop(0, NR // ZR, _zc, 0)
        plsc.subcore_barrier()

        def _chunk(ci, _):
            pltpu.sync_copy(e_hbm.at[c, s, ci], eb)
            pltpu.sync_copy(w_hbm.at[c, s, ci], wb)
            pltpu.async_copy(gather_ref.at[eb.at[0]], rows, sem).wait()

            def _scale(j, _):
                wv = wb[pl.ds(j * 16, 16)]
                for l in range(16):
                    ws = wv[l]
                    row = j * 16 + l
                    for k in range(8):
                        sl = pl.ds(k * 16, 16)
                        rows[row, sl] = rows[row, sl] * ws
                return 0
            lax.fori_loop(0, CH // 16, _scale, 0)
            pltpu.sync_copy(rows, shared.at[eb.at[1]], add=True)
            return 0
        lax.fori_loop(0, NCH, _chunk, 0)
        plsc.subcore_barrier()

    for p in range(2):
        _sweep(h1s_hbm.at[c, p])

        def _ut(t, _):
            base = s * NR + t * ZR
            pltpu.sync_copy(shared.at[pl.ds(base, ZR)], zb)
            pltpu.sync_copy(h1s_hbm.at[c, p, pl.ds(base, ZR)], hrows)

            def _pw(g, _):
                dv = dinv_v[pl.ds(t * ZR + g * 16, 16)]
                for l in range(16):
                    dr = dv[l]
                    row = g * 16 + l
                    for k in range(8):
                        sl = pl.ds(k * 16, 16)
                        b1k = b1_v[pl.ds(p * 128 + k * 16, 16)]
                        val = (zb[row, sl] + hrows[row, sl]) * dr + b1k
                        hrows[row, sl] = jnp.maximum(val, 0.0) * dr
                return 0
            lax.fori_loop(0, ZR // 16, _pw, 0)
            pltpu.sync_copy(hrows, u_hbm.at[c, p, pl.ds(base, ZR)])
            return 0
        lax.fori_loop(0, NR // ZR, _ut, 0)
    plsc.subcore_barrier()

    for p in range(2):
        _sweep(u_hbm.at[c, p])

        def _vt(t, _):
            base = s * NR + t * ZR
            pltpu.sync_copy(shared.at[pl.ds(base, ZR)], zb)
            pltpu.sync_copy(zb, v_hbm.at[c, p, pl.ds(base, ZR)])
            return 0
        lax.fori_loop(0, NR // ZR, _vt, 0)


_conv_call = pl.kernel(
    _conv_body,
    out_type=(jax.ShapeDtypeStruct((2, 2, NP, 128), jnp.float32),
              jax.ShapeDtypeStruct((2, 2, NP, 128), jnp.float32)),
    mesh=_sc_mesh,
    scratch_types=[
        pltpu.VMEM((2, CH), jnp.int32),
        pltpu.VMEM((CH,), jnp.float32),
        pltpu.VMEM((CH, 128), jnp.float32),
        pltpu.VMEM((ZR, 128), jnp.float32),
        pltpu.VMEM((ZR, 128), jnp.float32),
        pltpu.VMEM((NR,), jnp.float32),
        pltpu.VMEM((256,), jnp.float32),
        pltpu.SemaphoreType.DMA,
        pltpu.VMEM_SHARED((NP, 128), jnp.float32),
    ],
)


def _tc1_body(x_ref, w_ref, deg_ref, h1s_ref, dinv_ref):
    deg = deg_ref[0, :, 0] + 1.0
    dinv = jnp.where(deg > 0, lax.rsqrt(jnp.maximum(deg, 1e-12)), 0.0)
    h = jnp.dot(x_ref[0], w_ref[0], preferred_element_type=jnp.float32)
    hs = h * dinv[:, None]
    h1s_ref[0, 0] = hs[:, :128]
    h1s_ref[0, 1] = hs[:, 128:]
    dinv_ref[0] = dinv[:, None]


def _tc1(x, w1, deg):
    return pl.pallas_call(
        _tc1_body,
        grid=(2, NP // BLKP),
        in_specs=[
            pl.BlockSpec((1, BLKP, 256), lambda g, i: (g, i, 0)),
            pl.BlockSpec((1, 256, 256), lambda g, i: (g, 0, 0)),
            pl.BlockSpec((1, BLKP, 1), lambda g, i: (g, i, 0)),
        ],
        out_specs=[
            pl.BlockSpec((1, 2, BLKP, 128), lambda g, i: (g, 0, i, 0)),
            pl.BlockSpec((1, BLKP, 1), lambda g, i: (g, i, 0)),
        ],
        out_shape=[
            jax.ShapeDtypeStruct((2, 2, NP, 128), jnp.float32),
            jax.ShapeDtypeStruct((2, NP, 1), jnp.float32),
        ],
    )(x, w1, deg)


def _tc3_body(u_ref, v_ref, dinv_ref, w2_ref, b2_ref, enc_ref):
    a = jnp.concatenate(
        [u_ref[0, 0] + v_ref[0, 0], u_ref[0, 1] + v_ref[0, 1]], axis=1)
    a = a * dinv_ref[0]
    enc_ref[0] = jnp.dot(a, w2_ref[0],
                         preferred_element_type=jnp.float32) + b2_ref[0]


def _tc3(u, v, dinv, w2, b2):
    return pl.pallas_call(
        _tc3_body,
        grid=(2, N // BLK),
        in_specs=[
            pl.BlockSpec((1, 2, BLK, 128), lambda g, i: (g, 0, i, 0)),
            pl.BlockSpec((1, 2, BLK, 128), lambda g, i: (g, 0, i, 0)),
            pl.BlockSpec((1, BLK, 1), lambda g, i: (g, i, 0)),
            pl.BlockSpec((1, 256, 128), lambda g, i: (g, 0, 0)),
            pl.BlockSpec((1, 1, 128), lambda g, i: (g, 0, 0)),
        ],
        out_specs=pl.BlockSpec((1, BLK, 128), lambda g, i: (g, i, 0)),
        out_shape=jax.ShapeDtypeStruct((2, N, 128), jnp.float32),
    )(u, v, dinv, w2, b2)


def _dec_body(a_ref, b_ref, o_ref):
    z = lax.dot_general(a_ref[0], b_ref[0], (((1,), (1,)), ((), ())),
                        preferred_element_type=jnp.float32)
    o_ref[...] = jax.nn.sigmoid(z)


def _dec(enc):
    bj = 1280
    return pl.pallas_call(
        _dec_body,
        grid=(N // BLK, pl.cdiv(N, bj)),
        in_specs=[
            pl.BlockSpec((1, BLK, 128), lambda i, j: (0, i, 0)),
            pl.BlockSpec((1, bj, 128), lambda i, j: (1, j, 0)),
        ],
        out_specs=pl.BlockSpec((BLK, bj), lambda i, j: (i, j)),
        out_shape=jax.ShapeDtypeStruct((N, N), jnp.float32),
    )(enc, enc)


def kernel(nc_x, nc_edge_index, nc_edge_weight, dis_x, dis_edge_index,
           dis_edge_weight, NW1, Nb1, NW2, Nb2, DW1, Db1, DW2, Db2):
    src = jnp.stack([nc_edge_index[0], dis_edge_index[0]]
                    ).astype(jnp.int32).reshape(2, NS, NCH, CH)
    dst = jnp.stack([nc_edge_index[1], dis_edge_index[1]]
                    ).astype(jnp.int32).reshape(2, NS, NCH, CH)
    w = jnp.stack([nc_edge_weight, dis_edge_weight]).reshape(2, NS, NCH, CH)
    edges = jnp.stack([src, dst], axis=3)
    x = jnp.stack([nc_x, dis_x])
    w1 = jnp.stack([NW1, DW1])
    b1 = jnp.stack([Nb1, Db1])
    w2 = jnp.stack([NW2, DW2])
    b2 = jnp.stack([Nb2, Db2])[:, None, :]

    deg = _deg_call(edges, w)
    h1s, dinv = _tc1(x, w1, deg[:, :, None])
    u, v = _conv_call(h1s, edges, w, dinv[:, :, 0], b1)
    enc = _tc3(u, v, dinv, w2, b2)
    return _dec(enc)


# speedup vs baseline: 5.9363x; 1.3378x over previous
import jax
import jax.numpy as jnp
from jax import lax
from jax.experimental import pallas as pl
from jax.experimental.pallas import tpu as pltpu
from jax.experimental.pallas import tpu_sc as plsc

N = 10000
NP = 10240
E = 160000
NS = 16
ET = E // NS
CH = 80
NCH = ET // CH
NR = NP // NS
ZR = 32
BLK = 1000
BLKP = 1024

_sc_mesh = plsc.VectorSubcoreMesh(
    core_axis_name="c", subcore_axis_name="s", num_cores=2, num_subcores=16)


def _deg_body(e_hbm, w_hbm, deg_hbm, eb, wb, zb, shared):
    c = lax.axis_index("c")
    s = lax.axis_index("s")

    def _z(i, _):
        zb[pl.ds(i * 16, 16)] = jnp.zeros((16,), jnp.float32)
        return 0
    lax.fori_loop(0, NR // 16, _z, 0)
    pltpu.sync_copy(zb, shared.at[pl.ds(s * NR, NR)])
    plsc.subcore_barrier()

    def _acc(ci, _):
        pltpu.sync_copy(e_hbm.at[c, s, ci], eb)
        pltpu.sync_copy(w_hbm.at[c, s, ci], wb)
        pltpu.sync_copy(wb, shared.at[eb.at[1]], add=True)
        return 0
    lax.fori_loop(0, NCH, _acc, 0)
    plsc.subcore_barrier()

    pltpu.sync_copy(shared.at[pl.ds(s * NR, NR)], zb)
    pltpu.sync_copy(zb, deg_hbm.at[c, pl.ds(s * NR, NR)])


_deg_call = pl.kernel(
    _deg_body,
    out_type=jax.ShapeDtypeStruct((2, NP), jnp.float32),
    mesh=_sc_mesh,
    scratch_types=[
        pltpu.VMEM((2, CH), jnp.int32),
        pltpu.VMEM((CH,), jnp.float32),
        pltpu.VMEM((NR,), jnp.float32),
        pltpu.VMEM_SHARED((NP,), jnp.float32),
    ],
)


def _conv_body(h1s_hbm, e_hbm, w_hbm, dinv_hbm, b1_hbm, u_hbm, v_hbm,
               ebA, ebB, wbA, wbB, rows0, rows1, zb, hrows, dinv_v, b1_v,
               sem0, sem1, shared):
    c = lax.axis_index("c")
    s = lax.axis_index("s")
    pltpu.sync_copy(dinv_hbm.at[c, pl.ds(s * NR, NR)], dinv_v)
    pltpu.sync_copy(b1_hbm.at[c], b1_v)

    def _scale(rows, wb):
        def _sc(j, _):
            wv = wb[pl.ds(j * 16, 16)]
            for l in range(16):
                ws = wv[l]
                row = j * 16 + l
                for k in range(8):
                    sl = pl.ds(k * 16, 16)
                    rows[row, sl] = rows[row, sl] * ws
            return 0
        lax.fori_loop(0, CH // 16, _sc, 0)

    def _sweep(gather_ref):
        def _z(i, _):
            for k in range(8):
                zb[i, pl.ds(k * 16, 16)] = jnp.zeros((16,), jnp.float32)
            return 0
        lax.fori_loop(0, ZR, _z, 0)

        def _zc(t, _):
            pltpu.sync_copy(zb, shared.at[pl.ds(s * NR + t * ZR, ZR)])
            return 0
        lax.fori_loop(0, NR // ZR, _zc, 0)
        plsc.subcore_barrier()

        pltpu.sync_copy(e_hbm.at[c, s, 0], ebA)
        pltpu.sync_copy(w_hbm.at[c, s, 0], wbA)
        pltpu.async_copy(gather_ref.at[ebA.at[0]], rows0, sem0)

        def _pair(i, _):
            c0 = 2 * i
            pltpu.sync_copy(e_hbm.at[c, s, c0 + 1], ebB)
            pltpu.sync_copy(w_hbm.at[c, s, c0 + 1], wbB)
            pltpu.async_copy(gather_ref.at[ebB.at[0]], rows1, sem1)
            pltpu.make_async_copy(gather_ref.at[ebA.at[0]], rows0, sem0).wait()
            _scale(rows0, wbA)
            pltpu.sync_copy(rows0, shared.at[ebA.at[1]], add=True)

            @pl.when(c0 + 2 < NCH)
            def _():
                pltpu.sync_copy(e_hbm.at[c, s, c0 + 2], ebA)
                pltpu.sync_copy(w_hbm.at[c, s, c0 + 2], wbA)
                pltpu.async_copy(gather_ref.at[ebA.at[0]], rows0, sem0)
            pltpu.make_async_copy(gather_ref.at[ebB.at[0]], rows1, sem1).wait()
            _scale(rows1, wbB)
            pltpu.sync_copy(rows1, shared.at[ebB.at[1]], add=True)
            return 0
        lax.fori_loop(0, NCH // 2, _pair, 0)

        pltpu.make_async_copy(gather_ref.at[ebA.at[0]], rows0, sem0).wait()
        _scale(rows0, wbA)
        pltpu.sync_copy(rows0, shared.at[ebA.at[1]], add=True)
        plsc.subcore_barrier()

    def _p1(p, _):
        _sweep(h1s_hbm.at[c, p])

        def _ut(t, _):
            base = s * NR + t * ZR
            pltpu.sync_copy(shared.at[pl.ds(base, ZR)], zb)
            pltpu.sync_copy(h1s_hbm.at[c, p, pl.ds(base, ZR)], hrows)

            def _pw(g, _):
                dv = dinv_v[pl.ds(t * ZR + g * 16, 16)]
                for l in range(16):
                    dr = dv[l]
                    row = g * 16 + l
                    for k in range(8):
                        sl = pl.ds(k * 16, 16)
                        b1k = b1_v[pl.ds(p * 128 + k * 16, 16)]
                        val = (zb[row, sl] + hrows[row, sl]) * dr + b1k
                        hrows[row, sl] = jnp.maximum(val, 0.0) * dr
                return 0
            lax.fori_loop(0, ZR // 16, _pw, 0)
            pltpu.sync_copy(hrows, u_hbm.at[c, p, pl.ds(base, ZR)])
            return 0
        lax.fori_loop(0, NR // ZR, _ut, 0)
        return 0
    lax.fori_loop(0, 2, _p1, 0)
    plsc.subcore_barrier()

    def _p2(p, _):
        _sweep(u_hbm.at[c, p])

        def _vt(t, _):
            base = s * NR + t * ZR
            pltpu.sync_copy(shared.at[pl.ds(base, ZR)], zb)
            pltpu.sync_copy(zb, v_hbm.at[c, p, pl.ds(base, ZR)])
            return 0
        lax.fori_loop(0, NR // ZR, _vt, 0)
        return 0
    lax.fori_loop(0, 2, _p2, 0)


_conv_call = pl.kernel(
    _conv_body,
    out_type=(jax.ShapeDtypeStruct((2, 2, NP, 128), jnp.float32),
              jax.ShapeDtypeStruct((2, 2, NP, 128), jnp.float32)),
    mesh=_sc_mesh,
    scratch_types=[
        pltpu.VMEM((2, CH), jnp.int32),
        pltpu.VMEM((2, CH), jnp.int32),
        pltpu.VMEM((CH,), jnp.float32),
        pltpu.VMEM((CH,), jnp.float32),
        pltpu.VMEM((CH, 128), jnp.float32),
        pltpu.VMEM((CH, 128), jnp.float32),
        pltpu.VMEM((ZR, 128), jnp.float32),
        pltpu.VMEM((ZR, 128), jnp.float32),
        pltpu.VMEM((NR,), jnp.float32),
        pltpu.VMEM((256,), jnp.float32),
        pltpu.SemaphoreType.DMA,
        pltpu.SemaphoreType.DMA,
        pltpu.VMEM_SHARED((NP, 128), jnp.float32),
    ],
)


def _tc1_body(x_ref, w_ref, deg_ref, h1s_ref, dinv_ref):
    deg = deg_ref[0, :, 0] + 1.0
    dinv = jnp.where(deg > 0, lax.rsqrt(jnp.maximum(deg, 1e-12)), 0.0)
    h = jnp.dot(x_ref[0], w_ref[0], preferred_element_type=jnp.float32)
    hs = h * dinv[:, None]
    h1s_ref[0, 0] = hs[:, :128]
    h1s_ref[0, 1] = hs[:, 128:]
    dinv_ref[0] = dinv[:, None]


def _tc1(x, w1, deg):
    return pl.pallas_call(
        _tc1_body,
        grid=(2, NP // BLKP),
        in_specs=[
            pl.BlockSpec((1, BLKP, 256), lambda g, i: (g, i, 0)),
            pl.BlockSpec((1, 256, 256), lambda g, i: (g, 0, 0)),
            pl.BlockSpec((1, BLKP, 1), lambda g, i: (g, i, 0)),
        ],
        out_specs=[
            pl.BlockSpec((1, 2, BLKP, 128), lambda g, i: (g, 0, i, 0)),
            pl.BlockSpec((1, BLKP, 1), lambda g, i: (g, i, 0)),
        ],
        out_shape=[
            jax.ShapeDtypeStruct((2, 2, NP, 128), jnp.float32),
            jax.ShapeDtypeStruct((2, NP, 1), jnp.float32),
        ],
    )(x, w1, deg)


def _tc3_body(u_ref, v_ref, dinv_ref, w2_ref, b2_ref, enc_ref):
    a = jnp.concatenate(
        [u_ref[0, 0] + v_ref[0, 0], u_ref[0, 1] + v_ref[0, 1]], axis=1)
    a = a * dinv_ref[0]
    enc_ref[0] = jnp.dot(a, w2_ref[0],
                         preferred_element_type=jnp.float32) + b2_ref[0]


def _tc3(u, v, dinv, w2, b2):
    return pl.pallas_call(
        _tc3_body,
        grid=(2, N // BLK),
        in_specs=[
            pl.BlockSpec((1, 2, BLK, 128), lambda g, i: (g, 0, i, 0)),
            pl.BlockSpec((1, 2, BLK, 128), lambda g, i: (g, 0, i, 0)),
            pl.BlockSpec((1, BLK, 1), lambda g, i: (g, i, 0)),
            pl.BlockSpec((1, 256, 128), lambda g, i: (g, 0, 0)),
            pl.BlockSpec((1, 1, 128), lambda g, i: (g, 0, 0)),
        ],
        out_specs=pl.BlockSpec((1, BLK, 128), lambda g, i: (g, i, 0)),
        out_shape=jax.ShapeDtypeStruct((2, N, 128), jnp.float32),
    )(u, v, dinv, w2, b2)


def _dec_body(a_ref, b_ref, o_ref):
    z = lax.dot_general(a_ref[0], b_ref[0], (((1,), (1,)), ((), ())),
                        preferred_element_type=jnp.float32)
    o_ref[...] = jax.nn.sigmoid(z)


def _dec(enc):
    bj = 1280
    return pl.pallas_call(
        _dec_body,
        grid=(N // BLK, pl.cdiv(N, bj)),
        in_specs=[
            pl.BlockSpec((1, BLK, 128), lambda i, j: (0, i, 0)),
            pl.BlockSpec((1, bj, 128), lambda i, j: (1, j, 0)),
        ],
        out_specs=pl.BlockSpec((BLK, bj), lambda i, j: (i, j)),
        out_shape=jax.ShapeDtypeStruct((N, N), jnp.float32),
    )(enc, enc)


def kernel(nc_x, nc_edge_index, nc_edge_weight, dis_x, dis_edge_index,
           dis_edge_weight, NW1, Nb1, NW2, Nb2, DW1, Db1, DW2, Db2):
    src = jnp.stack([nc_edge_index[0], dis_edge_index[0]]
                    ).astype(jnp.int32).reshape(2, NS, NCH, CH)
    dst = jnp.stack([nc_edge_index[1], dis_edge_index[1]]
                    ).astype(jnp.int32).reshape(2, NS, NCH, CH)
    w = jnp.stack([nc_edge_weight, dis_edge_weight]).reshape(2, NS, NCH, CH)
    edges = jnp.stack([src, dst], axis=3)
    x = jnp.stack([nc_x, dis_x])
    w1 = jnp.stack([NW1, DW1])
    b1 = jnp.stack([Nb1, Db1])
    w2 = jnp.stack([NW2, DW2])
    b2 = jnp.stack([Nb2, Db2])[:, None, :]

    deg = _deg_call(edges, w)
    h1s, dinv = _tc1(x, w1, deg[:, :, None])
    u, v = _conv_call(h1s, edges, w, dinv[:, :, 0], b1)
    enc = _tc3(u, v, dinv, w2, b2)
    return _dec(enc)


# speedup vs baseline: 7.3336x; 1.2354x over previous
import jax
import jax.numpy as jnp
from jax import lax
from jax.experimental import pallas as pl
from jax.experimental.pallas import tpu as pltpu
from jax.experimental.pallas import tpu_sc as plsc

N = 10000
NP = 10240
E = 160000
NS = 16
ET = E // NS
CH = 80
NCH = ET // CH
NR = NP // NS
ZR = 32
BLK = 1000
BLKP = 1024

_sc_mesh = plsc.VectorSubcoreMesh(
    core_axis_name="c", subcore_axis_name="s", num_cores=2, num_subcores=16)


def _deg_body(e_hbm, w_hbm, deg_hbm, eb, wb, zb, shared):
    c = lax.axis_index("c")
    s = lax.axis_index("s")

    def _z(i, _):
        zb[pl.ds(i * 16, 16)] = jnp.zeros((16,), jnp.float32)
        return 0
    lax.fori_loop(0, NR // 16, _z, 0)
    pltpu.sync_copy(zb, shared.at[pl.ds(s * NR, NR)])
    plsc.subcore_barrier()

    def _acc(ci, _):
        pltpu.sync_copy(e_hbm.at[c, s, ci], eb)
        pltpu.sync_copy(w_hbm.at[c, s, ci, 0], wb)
        pltpu.sync_copy(wb, shared.at[eb.at[1]], add=True)
        return 0
    lax.fori_loop(0, NCH, _acc, 0)
    plsc.subcore_barrier()

    pltpu.sync_copy(shared.at[pl.ds(s * NR, NR)], zb)
    pltpu.sync_copy(zb, deg_hbm.at[c, pl.ds(s * NR, NR)])


_deg_call = pl.kernel(
    _deg_body,
    out_type=jax.ShapeDtypeStruct((2, NP), jnp.float32),
    mesh=_sc_mesh,
    scratch_types=[
        pltpu.VMEM((2, CH), jnp.int32),
        pltpu.VMEM((CH,), jnp.float32),
        pltpu.VMEM((NR,), jnp.float32),
        pltpu.VMEM_SHARED((NP,), jnp.float32),
    ],
)


def _conv_body(h1s_hbm, e_hbm, w_hbm, dinv_hbm, b1_hbm, u_hbm, v_hbm,
               ebPA, ebPB, wbPA, wbPB, r0, r1, r2, r3, dinv_v, b1_v,
               g0, g1, g2, g3, s0, s1, s2, s3, shared):
    c = lax.axis_index("c")
    s = lax.axis_index("s")
    pltpu.sync_copy(dinv_hbm.at[c, pl.ds(s * NR, NR)], dinv_v)
    pltpu.sync_copy(b1_hbm.at[c], b1_v)

    def _scale(rows, wbP, k):
        def _sc(j, _):
            wv = wbP[k, 0, pl.ds(j * 16, 16)]
            for l in range(16):
                ws = wv[l]
                row = j * 16 + l
                for kk in range(8):
                    sl = pl.ds(kk * 16, 16)
                    rows[row, sl] = rows[row, sl] * ws
            return 0
        lax.fori_loop(0, CH // 16, _sc, 0)

    def _sweep(gather_ref):
        def _z(i, _):
            for k in range(8):
                r1[i, pl.ds(k * 16, 16)] = jnp.zeros((16,), jnp.float32)
            return 0
        lax.fori_loop(0, CH, _z, 0)

        def _zc(t, _):
            pltpu.sync_copy(r1, shared.at[pl.ds(s * NR + t * CH, CH)])
            return 0
        lax.fori_loop(0, NR // CH, _zc, 0)
        plsc.subcore_barrier()

        pltpu.sync_copy(e_hbm.at[c, s, pl.ds(0, 2)], ebPA)
        pltpu.sync_copy(w_hbm.at[c, s, pl.ds(0, 2)], wbPA)
        pltpu.async_copy(gather_ref.at[ebPA.at[0, 0]], r0, g0)
        pltpu.async_copy(gather_ref.at[ebPA.at[1, 0]], r1, g1)

        def _quad(ii, _):
            c0 = 4 * ii

            @pl.when(ii > 0)
            def _():
                pltpu.make_async_copy(r2, shared.at[ebPB.at[0, 1]], s2).wait()
                pltpu.make_async_copy(r3, shared.at[ebPB.at[1, 1]], s3).wait()
            pltpu.sync_copy(e_hbm.at[c, s, pl.ds(c0 + 2, 2)], ebPB)
            pltpu.sync_copy(w_hbm.at[c, s, pl.ds(c0 + 2, 2)], wbPB)
            pltpu.async_copy(gather_ref.at[ebPB.at[0, 0]], r2, g2)
            pltpu.async_copy(gather_ref.at[ebPB.at[1, 0]], r3, g3)

            pltpu.make_async_copy(gather_ref.at[ebPA.at[0, 0]], r0, g0).wait()
            _scale(r0, wbPA, 0)
            pltpu.async_copy(r0, shared.at[ebPA.at[0, 1]], s0, add=True)
            pltpu.make_async_copy(gather_ref.at[ebPA.at[1, 0]], r1, g1).wait()
            _scale(r1, wbPA, 1)
            pltpu.async_copy(r1, shared.at[ebPA.at[1, 1]], s1, add=True)

            pltpu.make_async_copy(r0, shared.at[ebPA.at[0, 1]], s0).wait()
            pltpu.make_async_copy(r1, shared.at[ebPA.at[1, 1]], s1).wait()

            @pl.when(ii < NCH // 4 - 1)
            def _():
                pltpu.sync_copy(e_hbm.at[c, s, pl.ds(c0 + 4, 2)], ebPA)
                pltpu.sync_copy(w_hbm.at[c, s, pl.ds(c0 + 4, 2)], wbPA)
                pltpu.async_copy(gather_ref.at[ebPA.at[0, 0]], r0, g0)
                pltpu.async_copy(gather_ref.at[ebPA.at[1, 0]], r1, g1)

            pltpu.make_async_copy(gather_ref.at[ebPB.at[0, 0]], r2, g2).wait()
            _scale(r2, wbPB, 0)
            pltpu.async_copy(r2, shared.at[ebPB.at[0, 1]], s2, add=True)
            pltpu.make_async_copy(gather_ref.at[ebPB.at[1, 0]], r3, g3).wait()
            _scale(r3, wbPB, 1)
            pltpu.async_copy(r3, shared.at[ebPB.at[1, 1]], s3, add=True)
            return 0
        lax.fori_loop(0, NCH // 4, _quad, 0)
        pltpu.make_async_copy(r2, shared.at[ebPB.at[0, 1]], s2).wait()
        pltpu.make_async_copy(r3, shared.at[ebPB.at[1, 1]], s3).wait()

        pltpu.sync_copy(e_hbm.at[c, s, pl.ds(NCH - 1, 1)], ebPA.at[pl.ds(0, 1)])
        pltpu.sync_copy(w_hbm.at[c, s, pl.ds(NCH - 1, 1)], wbPA.at[pl.ds(0, 1)])
        pltpu.async_copy(gather_ref.at[ebPA.at[0, 0]], r0, g0)
        pltpu.make_async_copy(gather_ref.at[ebPA.at[0, 0]], r0, g0).wait()
        _scale(r0, wbPA, 0)
        pltpu.sync_copy(r0, shared.at[ebPA.at[0, 1]], add=True)
        plsc.subcore_barrier()

    def _p1(p, _):
        _sweep(h1s_hbm.at[c, p])

        def _ut(t, _):
            base = s * NR + t * ZR
            pltpu.sync_copy(shared.at[pl.ds(base, ZR)], r2.at[pl.ds(0, ZR)])
            pltpu.sync_copy(h1s_hbm.at[c, p, pl.ds(base, ZR)],
                            r3.at[pl.ds(0, ZR)])

            def _pw(g, _):
                dv = dinv_v[pl.ds(t * ZR + g * 16, 16)]
                for l in range(16):
                    dr = dv[l]
                    row = g * 16 + l
                    for k in range(8):
                        sl = pl.ds(k * 16, 16)
                        b1k = b1_v[pl.ds(p * 128 + k * 16, 16)]
                        val = (r2[row, sl] + r3[row, sl]) * dr + b1k
                        r3[row, sl] = jnp.maximum(val, 0.0) * dr
                return 0
            lax.fori_loop(0, ZR // 16, _pw, 0)
            pltpu.sync_copy(r3.at[pl.ds(0, ZR)], u_hbm.at[c, p, pl.ds(base, ZR)])
            return 0
        lax.fori_loop(0, NR // ZR, _ut, 0)
        return 0
    lax.fori_loop(0, 2, _p1, 0)
    plsc.subcore_barrier()

    def _p2(p, _):
        _sweep(u_hbm.at[c, p])

        def _vt(t, _):
            base = s * NR + t * ZR
            pltpu.sync_copy(shared.at[pl.ds(base, ZR)], r2.at[pl.ds(0, ZR)])
            pltpu.sync_copy(r2.at[pl.ds(0, ZR)], v_hbm.at[c, p, pl.ds(base, ZR)])
            return 0
        lax.fori_loop(0, NR // ZR, _vt, 0)
        return 0
    lax.fori_loop(0, 2, _p2, 0)


_conv_call = pl.kernel(
    _conv_body,
    out_type=(jax.ShapeDtypeStruct((2, 2, NP, 128), jnp.float32),
              jax.ShapeDtypeStruct((2, 2, NP, 128), jnp.float32)),
    mesh=_sc_mesh,
    scratch_types=[
        pltpu.VMEM((2, 2, CH), jnp.int32),
        pltpu.VMEM((2, 2, CH), jnp.int32),
        pltpu.VMEM((2, 1, CH), jnp.float32),
        pltpu.VMEM((2, 1, CH), jnp.float32),
        pltpu.VMEM((CH, 128), jnp.float32),
        pltpu.VMEM((CH, 128), jnp.float32),
        pltpu.VMEM((CH, 128), jnp.float32),
        pltpu.VMEM((CH, 128), jnp.float32),
        pltpu.VMEM((NR,), jnp.float32),
        pltpu.VMEM((256,), jnp.float32),
        pltpu.SemaphoreType.DMA,
        pltpu.SemaphoreType.DMA,
        pltpu.SemaphoreType.DMA,
        pltpu.SemaphoreType.DMA,
        pltpu.SemaphoreType.DMA,
        pltpu.SemaphoreType.DMA,
        pltpu.SemaphoreType.DMA,
        pltpu.SemaphoreType.DMA,
        pltpu.VMEM_SHARED((NP, 128), jnp.float32),
    ],
)


def _tc1_body(x_ref, w_ref, deg_ref, h1s_ref, dinv_ref):
    deg = deg_ref[0, :, 0] + 1.0
    dinv = jnp.where(deg > 0, lax.rsqrt(jnp.maximum(deg, 1e-12)), 0.0)
    h = jnp.dot(x_ref[0], w_ref[0], preferred_element_type=jnp.float32)
    hs = h * dinv[:, None]
    h1s_ref[0, 0] = hs[:, :128]
    h1s_ref[0, 1] = hs[:, 128:]
    dinv_ref[0] = dinv[:, None]


def _tc1(x, w1, deg):
    return pl.pallas_call(
        _tc1_body,
        grid=(2, NP // BLKP),
        in_specs=[
            pl.BlockSpec((1, BLKP, 256), lambda g, i: (g, i, 0)),
            pl.BlockSpec((1, 256, 256), lambda g, i: (g, 0, 0)),
            pl.BlockSpec((1, BLKP, 1), lambda g, i: (g, i, 0)),
        ],
        out_specs=[
            pl.BlockSpec((1, 2, BLKP, 128), lambda g, i: (g, 0, i, 0)),
            pl.BlockSpec((1, BLKP, 1), lambda g, i: (g, i, 0)),
        ],
        out_shape=[
            jax.ShapeDtypeStruct((2, 2, NP, 128), jnp.float32),
            jax.ShapeDtypeStruct((2, NP, 1), jnp.float32),
        ],
    )(x, w1, deg)


def _tc3_body(u_ref, v_ref, dinv_ref, w2_ref, b2_ref, enc_ref):
    a = jnp.concatenate(
        [u_ref[0, 0] + v_ref[0, 0], u_ref[0, 1] + v_ref[0, 1]], axis=1)
    a = a * dinv_ref[0]
    enc_ref[0] = jnp.dot(a, w2_ref[0],
                         preferred_element_type=jnp.float32) + b2_ref[0]


def _tc3(u, v, dinv, w2, b2):
    return pl.pallas_call(
        _tc3_body,
        grid=(2, N // BLK),
        in_specs=[
            pl.BlockSpec((1, 2, BLK, 128), lambda g, i: (g, 0, i, 0)),
            pl.BlockSpec((1, 2, BLK, 128), lambda g, i: (g, 0, i, 0)),
            pl.BlockSpec((1, BLK, 1), lambda g, i: (g, i, 0)),
            pl.BlockSpec((1, 256, 128), lambda g, i: (g, 0, 0)),
            pl.BlockSpec((1, 1, 128), lambda g, i: (g, 0, 0)),
        ],
        out_specs=pl.BlockSpec((1, BLK, 128), lambda g, i: (g, i, 0)),
        out_shape=jax.ShapeDtypeStruct((2, N, 128), jnp.float32),
    )(u, v, dinv, w2, b2)


def _dec_body(a_ref, b_ref, o_ref):
    z = lax.dot_general(a_ref[0], b_ref[0], (((1,), (1,)), ((), ())),
                        preferred_element_type=jnp.float32)
    o_ref[...] = jax.nn.sigmoid(z)


def _dec(enc):
    bj = 1280
    return pl.pallas_call(
        _dec_body,
        grid=(N // BLK, pl.cdiv(N, bj)),
        in_specs=[
            pl.BlockSpec((1, BLK, 128), lambda i, j: (0, i, 0)),
            pl.BlockSpec((1, bj, 128), lambda i, j: (1, j, 0)),
        ],
        out_specs=pl.BlockSpec((BLK, bj), lambda i, j: (i, j)),
        out_shape=jax.ShapeDtypeStruct((N, N), jnp.float32),
    )(enc, enc)


def kernel(nc_x, nc_edge_index, nc_edge_weight, dis_x, dis_edge_index,
           dis_edge_weight, NW1, Nb1, NW2, Nb2, DW1, Db1, DW2, Db2):
    src = jnp.stack([nc_edge_index[0], dis_edge_index[0]]
                    ).astype(jnp.int32).reshape(2, NS, NCH, CH)
    dst = jnp.stack([nc_edge_index[1], dis_edge_index[1]]
                    ).astype(jnp.int32).reshape(2, NS, NCH, CH)
    w = jnp.stack([nc_edge_weight, dis_edge_weight]).reshape(2, NS, NCH, 1, CH)
    edges = jnp.stack([src, dst], axis=3)
    x = jnp.stack([nc_x, dis_x])
    w1 = jnp.stack([NW1, DW1])
    b1 = jnp.stack([Nb1, Db1])
    w2 = jnp.stack([NW2, DW2])
    b2 = jnp.stack([Nb2, Db2])[:, None, :]

    deg = _deg_call(edges, w)
    h1s, dinv = _tc1(x, w1, deg[:, :, None])
    u, v = _conv_call(h1s, edges, w, dinv[:, :, 0], b1)
    enc = _tc3(u, v, dinv, w2, b2)
    return _dec(enc)


# speedup vs baseline: 8.0714x; 1.1006x over previous
import jax
import jax.numpy as jnp
from jax import lax
from jax.experimental import pallas as pl
from jax.experimental.pallas import tpu as pltpu
from jax.experimental.pallas import tpu_sc as plsc

N = 10000
NP = 10240
E = 160000
NS = 16
ET = E // NS
CH = 80
NCH = ET // CH
NR = NP // NS
ZR = 80
BLK = 1000
BLKP = 1024

_sc_mesh = plsc.VectorSubcoreMesh(
    core_axis_name="c", subcore_axis_name="s", num_cores=2, num_subcores=16)


def _deg_body(e_hbm, w_hbm, deg_hbm, eb, wb, zb, shared):
    c = lax.axis_index("c")
    s = lax.axis_index("s")

    def _z(i, _):
        zb[pl.ds(i * 16, 16)] = jnp.zeros((16,), jnp.float32)
        return 0
    lax.fori_loop(0, NR // 16, _z, 0)
    pltpu.sync_copy(zb, shared.at[pl.ds(s * NR, NR)])
    plsc.subcore_barrier()

    def _acc(gi, _):
        pltpu.sync_copy(e_hbm.at[c, s, pl.ds(gi * 5, 5)], eb)
        pltpu.sync_copy(w_hbm.at[c, s, pl.ds(gi * 5, 5)], wb)
        for q in range(5):
            pltpu.sync_copy(wb.at[q, 0], shared.at[eb.at[q, 1]], add=True)
        return 0
    lax.fori_loop(0, NCH // 5, _acc, 0)
    plsc.subcore_barrier()

    pltpu.sync_copy(shared.at[pl.ds(s * NR, NR)], zb)
    pltpu.sync_copy(zb, deg_hbm.at[c, pl.ds(s * NR, NR)])


_deg_call = pl.kernel(
    _deg_body,
    out_type=jax.ShapeDtypeStruct((2, NP), jnp.float32),
    mesh=_sc_mesh,
    scratch_types=[
        pltpu.VMEM((5, 2, CH), jnp.int32),
        pltpu.VMEM((5, 1, CH), jnp.float32),
        pltpu.VMEM((NR,), jnp.float32),
        pltpu.VMEM_SHARED((NP,), jnp.float32),
    ],
)


def _conv_body(h1s_hbm, e_hbm, w_hbm, dinv_hbm, b1_hbm, u_hbm, v_hbm,
               ebPA, ebPB, wbPA, wbPB, r0, r1, r2, r3, dinv_v, b1_v,
               g0, g1, g2, g3, s0, s1, s2, s3, shared):
    c = lax.axis_index("c")
    s = lax.axis_index("s")
    pltpu.sync_copy(dinv_hbm.at[c, pl.ds(s * NR, NR)], dinv_v)
    pltpu.sync_copy(b1_hbm.at[c], b1_v)

    def _scale(rows, wbP, k):
        def _sc(j, _):
            wv = wbP[k, 0, pl.ds(j * 16, 16)]
            for l in range(16):
                ws = wv[l]
                row = j * 16 + l
                for kk in range(8):
                    sl = pl.ds(kk * 16, 16)
                    rows[row, sl] = rows[row, sl] * ws
            return 0
        lax.fori_loop(0, CH // 16, _sc, 0)

    def _sweep(gather_ref):
        def _z(i, _):
            for k in range(8):
                r1[i, pl.ds(k * 16, 16)] = jnp.zeros((16,), jnp.float32)
            return 0
        lax.fori_loop(0, CH, _z, 0)

        def _zc(t, _):
            pltpu.sync_copy(r1, shared.at[pl.ds(s * NR + t * CH, CH)])
            return 0
        lax.fori_loop(0, NR // CH, _zc, 0)
        plsc.subcore_barrier()

        pltpu.sync_copy(e_hbm.at[c, s, pl.ds(0, 2)], ebPA)
        pltpu.sync_copy(w_hbm.at[c, s, pl.ds(0, 2)], wbPA)
        pltpu.async_copy(gather_ref.at[ebPA.at[0, 0]], r0, g0)
        pltpu.async_copy(gather_ref.at[ebPA.at[1, 0]], r1, g1)

        def _quad(ii, _):
            c0 = 4 * ii

            @pl.when(ii > 0)
            def _():
                pltpu.make_async_copy(r2, shared.at[ebPB.at[0, 1]], s2).wait()
                pltpu.make_async_copy(r3, shared.at[ebPB.at[1, 1]], s3).wait()
            pltpu.sync_copy(e_hbm.at[c, s, pl.ds(c0 + 2, 2)], ebPB)
            pltpu.sync_copy(w_hbm.at[c, s, pl.ds(c0 + 2, 2)], wbPB)
            pltpu.async_copy(gather_ref.at[ebPB.at[0, 0]], r2, g2)
            pltpu.async_copy(gather_ref.at[ebPB.at[1, 0]], r3, g3)

            pltpu.make_async_copy(gather_ref.at[ebPA.at[0, 0]], r0, g0).wait()
            _scale(r0, wbPA, 0)
            pltpu.async_copy(r0, shared.at[ebPA.at[0, 1]], s0, add=True)
            pltpu.make_async_copy(gather_ref.at[ebPA.at[1, 0]], r1, g1).wait()
            _scale(r1, wbPA, 1)
            pltpu.async_copy(r1, shared.at[ebPA.at[1, 1]], s1, add=True)

            pltpu.make_async_copy(r0, shared.at[ebPA.at[0, 1]], s0).wait()
            pltpu.make_async_copy(r1, shared.at[ebPA.at[1, 1]], s1).wait()

            @pl.when(ii < NCH // 4 - 1)
            def _():
                pltpu.sync_copy(e_hbm.at[c, s, pl.ds(c0 + 4, 2)], ebPA)
                pltpu.sync_copy(w_hbm.at[c, s, pl.ds(c0 + 4, 2)], wbPA)
                pltpu.async_copy(gather_ref.at[ebPA.at[0, 0]], r0, g0)
                pltpu.async_copy(gather_ref.at[ebPA.at[1, 0]], r1, g1)

            pltpu.make_async_copy(gather_ref.at[ebPB.at[0, 0]], r2, g2).wait()
            _scale(r2, wbPB, 0)
            pltpu.async_copy(r2, shared.at[ebPB.at[0, 1]], s2, add=True)
            pltpu.make_async_copy(gather_ref.at[ebPB.at[1, 0]], r3, g3).wait()
            _scale(r3, wbPB, 1)
            pltpu.async_copy(r3, shared.at[ebPB.at[1, 1]], s3, add=True)
            return 0
        lax.fori_loop(0, NCH // 4, _quad, 0)
        pltpu.make_async_copy(r2, shared.at[ebPB.at[0, 1]], s2).wait()
        pltpu.make_async_copy(r3, shared.at[ebPB.at[1, 1]], s3).wait()

        pltpu.sync_copy(e_hbm.at[c, s, pl.ds(NCH - 1, 1)], ebPA.at[pl.ds(0, 1)])
        pltpu.sync_copy(w_hbm.at[c, s, pl.ds(NCH - 1, 1)], wbPA.at[pl.ds(0, 1)])
        pltpu.async_copy(gather_ref.at[ebPA.at[0, 0]], r0, g0)
        pltpu.make_async_copy(gather_ref.at[ebPA.at[0, 0]], r0, g0).wait()
        _scale(r0, wbPA, 0)
        pltpu.sync_copy(r0, shared.at[ebPA.at[0, 1]], add=True)
        plsc.subcore_barrier()

    def _p1(p, _):
        _sweep(h1s_hbm.at[c, p])

        def _ut(t, _):
            base = s * NR + t * ZR
            pltpu.sync_copy(shared.at[pl.ds(base, ZR)], r2.at[pl.ds(0, ZR)])
            pltpu.sync_copy(h1s_hbm.at[c, p, pl.ds(base, ZR)],
                            r3.at[pl.ds(0, ZR)])

            def _pw(g, _):
                dv = dinv_v[pl.ds(t * ZR + g * 16, 16)]
                for l in range(16):
                    dr = dv[l]
                    row = g * 16 + l
                    for k in range(8):
                        sl = pl.ds(k * 16, 16)
                        b1k = b1_v[pl.ds(p * 128 + k * 16, 16)]
                        val = (r2[row, sl] + r3[row, sl]) * dr + b1k
                        r3[row, sl] = jnp.maximum(val, 0.0) * dr
                return 0
            lax.fori_loop(0, ZR // 16, _pw, 0)
            pltpu.sync_copy(r3.at[pl.ds(0, ZR)], u_hbm.at[c, p, pl.ds(base, ZR)])
            return 0
        lax.fori_loop(0, NR // ZR, _ut, 0)
        return 0
    lax.fori_loop(0, 2, _p1, 0)
    plsc.subcore_barrier()

    def _p2(p, _):
        _sweep(u_hbm.at[c, p])

        def _vt(t, _):
            base = s * NR + t * ZR
            pltpu.sync_copy(shared.at[pl.ds(base, ZR)], r2.at[pl.ds(0, ZR)])
            pltpu.sync_copy(r2.at[pl.ds(0, ZR)], v_hbm.at[c, p, pl.ds(base, ZR)])
            return 0
        lax.fori_loop(0, NR // ZR, _vt, 0)
        return 0
    lax.fori_loop(0, 2, _p2, 0)


_conv_call = pl.kernel(
    _conv_body,
    out_type=(jax.ShapeDtypeStruct((2, 2, NP, 128), jnp.float32),
              jax.ShapeDtypeStruct((2, 2, NP, 128), jnp.float32)),
    mesh=_sc_mesh,
    scratch_types=[
        pltpu.VMEM((2, 2, CH), jnp.int32),
        pltpu.VMEM((2, 2, CH), jnp.int32),
        pltpu.VMEM((2, 1, CH), jnp.float32),
        pltpu.VMEM((2, 1, CH), jnp.float32),
        pltpu.VMEM((CH, 128), jnp.float32),
        pltpu.VMEM((CH, 128), jnp.float32),
        pltpu.VMEM((CH, 128), jnp.float32),
        pltpu.VMEM((CH, 128), jnp.float32),
        pltpu.VMEM((NR,), jnp.float32),
        pltpu.VMEM((256,), jnp.float32),
        pltpu.SemaphoreType.DMA,
        pltpu.SemaphoreType.DMA,
        pltpu.SemaphoreType.DMA,
        pltpu.SemaphoreType.DMA,
        pltpu.SemaphoreType.DMA,
        pltpu.SemaphoreType.DMA,
        pltpu.SemaphoreType.DMA,
        pltpu.SemaphoreType.DMA,
        pltpu.VMEM_SHARED((NP, 128), jnp.float32),
    ],
)


def _tc1_body(x_ref, w_ref, deg_ref, h1s_ref, dinv_ref):
    deg = deg_ref[0, :, 0] + 1.0
    dinv = jnp.where(deg > 0, lax.rsqrt(jnp.maximum(deg, 1e-12)), 0.0)
    h = jnp.dot(x_ref[0], w_ref[0], preferred_element_type=jnp.float32)
    hs = h * dinv[:, None]
    h1s_ref[0, 0] = hs[:, :128]
    h1s_ref[0, 1] = hs[:, 128:]
    dinv_ref[0] = dinv[:, None]


def _tc1(x, w1, deg):
    return pl.pallas_call(
        _tc1_body,
        grid=(2, NP // BLKP),
        in_specs=[
            pl.BlockSpec((1, BLKP, 256), lambda g, i: (g, i, 0)),
            pl.BlockSpec((1, 256, 256), lambda g, i: (g, 0, 0)),
            pl.BlockSpec((1, BLKP, 1), lambda g, i: (g, i, 0)),
        ],
        out_specs=[
            pl.BlockSpec((1, 2, BLKP, 128), lambda g, i: (g, 0, i, 0)),
            pl.BlockSpec((1, BLKP, 1), lambda g, i: (g, i, 0)),
        ],
        out_shape=[
            jax.ShapeDtypeStruct((2, 2, NP, 128), jnp.float32),
            jax.ShapeDtypeStruct((2, NP, 1), jnp.float32),
        ],
    )(x, w1, deg)


def _tc3_body(u_ref, v_ref, dinv_ref, w2_ref, b2_ref, enc_ref):
    a = jnp.concatenate(
        [u_ref[0, 0] + v_ref[0, 0], u_ref[0, 1] + v_ref[0, 1]], axis=1)
    a = a * dinv_ref[0]
    enc_ref[0] = jnp.dot(a, w2_ref[0],
                         preferred_element_type=jnp.float32) + b2_ref[0]


def _tc3(u, v, dinv, w2, b2):
    return pl.pallas_call(
        _tc3_body,
        grid=(2, N // BLK),
        in_specs=[
            pl.BlockSpec((1, 2, BLK, 128), lambda g, i: (g, 0, i, 0)),
            pl.BlockSpec((1, 2, BLK, 128), lambda g, i: (g, 0, i, 0)),
            pl.BlockSpec((1, BLK, 1), lambda g, i: (g, i, 0)),
            pl.BlockSpec((1, 256, 128), lambda g, i: (g, 0, 0)),
            pl.BlockSpec((1, 1, 128), lambda g, i: (g, 0, 0)),
        ],
        out_specs=pl.BlockSpec((1, BLK, 128), lambda g, i: (g, i, 0)),
        out_shape=jax.ShapeDtypeStruct((2, N, 128), jnp.float32),
    )(u, v, dinv, w2, b2)


def _dec_body(a_ref, b_ref, o_ref):
    z = lax.dot_general(a_ref[0], b_ref[0], (((1,), (1,)), ((), ())),
                        preferred_element_type=jnp.float32)
    o_ref[...] = jax.nn.sigmoid(z)


def _dec(enc):
    bj = 1280
    return pl.pallas_call(
        _dec_body,
        grid=(N // BLK, pl.cdiv(N, bj)),
        in_specs=[
            pl.BlockSpec((1, BLK, 128), lambda i, j: (0, i, 0)),
            pl.BlockSpec((1, bj, 128), lambda i, j: (1, j, 0)),
        ],
        out_specs=pl.BlockSpec((BLK, bj), lambda i, j: (i, j)),
        out_shape=jax.ShapeDtypeStruct((N, N), jnp.float32),
    )(enc, enc)


def kernel(nc_x, nc_edge_index, nc_edge_weight, dis_x, dis_edge_index,
           dis_edge_weight, NW1, Nb1, NW2, Nb2, DW1, Db1, DW2, Db2):
    src = jnp.stack([nc_edge_index[0], dis_edge_index[0]]
                    ).astype(jnp.int32).reshape(2, NS, NCH, CH)
    dst = jnp.stack([nc_edge_index[1], dis_edge_index[1]]
                    ).astype(jnp.int32).reshape(2, NS, NCH, CH)
    w = jnp.stack([nc_edge_weight, dis_edge_weight]).reshape(2, NS, NCH, 1, CH)
    edges = jnp.stack([src, dst], axis=3)
    x = jnp.stack([nc_x, dis_x])
    w1 = jnp.stack([NW1, DW1])
    b1 = jnp.stack([Nb1, Db1])
    w2 = jnp.stack([NW2, DW2])
    b2 = jnp.stack([Nb2, Db2])[:, None, :]

    deg = _deg_call(edges, w)
    h1s, dinv = _tc1(x, w1, deg[:, :, None])
    u, v = _conv_call(h1s, edges, w, dinv[:, :, 0], b1)
    enc = _tc3(u, v, dinv, w2, b2)
    return _dec(enc)


# speedup vs baseline: 8.4274x; 1.0441x over previous
import jax
import jax.numpy as jnp
from jax import lax
from jax.experimental import pallas as pl
from jax.experimental.pallas import tpu as pltpu
from jax.experimental.pallas import tpu_sc as plsc

N = 10000
NP = 10240
E = 160000
NS = 16
ET = E // NS
CH = 80
NCH = ET // CH
NR = NP // NS
ZR = 80
BLK = 1000
BLKP = 1024

_sc_mesh = plsc.VectorSubcoreMesh(
    core_axis_name="c", subcore_axis_name="s", num_cores=2, num_subcores=16)


def _deg_body(e_hbm, w_hbm, deg_hbm, eb, wb, zb, shared):
    c = lax.axis_index("c")
    s = lax.axis_index("s")

    def _z(i, _):
        zb[pl.ds(i * 16, 16)] = jnp.zeros((16,), jnp.float32)
        return 0
    lax.fori_loop(0, NR // 16, _z, 0)
    pltpu.sync_copy(zb, shared.at[pl.ds(s * NR, NR)])
    plsc.subcore_barrier()

    def _acc(gi, _):
        pltpu.sync_copy(e_hbm.at[c, s, pl.ds(gi * 5, 5)], eb)
        pltpu.sync_copy(w_hbm.at[c, s, pl.ds(gi * 5, 5)], wb)
        for q in range(5):
            pltpu.sync_copy(wb.at[q, 0], shared.at[eb.at[q, 1]], add=True)
        return 0
    lax.fori_loop(0, NCH // 5, _acc, 0)
    plsc.subcore_barrier()

    pltpu.sync_copy(shared.at[pl.ds(s * NR, NR)], zb)
    pltpu.sync_copy(zb, deg_hbm.at[c, pl.ds(s * NR, NR)])


_deg_call = pl.kernel(
    _deg_body,
    out_type=jax.ShapeDtypeStruct((2, NP), jnp.float32),
    mesh=_sc_mesh,
    scratch_types=[
        pltpu.VMEM((5, 2, CH), jnp.int32),
        pltpu.VMEM((5, 1, CH), jnp.float32),
        pltpu.VMEM((NR,), jnp.float32),
        pltpu.VMEM_SHARED((NP,), jnp.float32),
    ],
)


def _conv_body(h1s_hbm, e_hbm, w_hbm, dinv_hbm, b1_hbm, u_hbm, v_hbm,
               ebPA, ebPB, wbPA, wbPB, r0, r1, r2, r3, dinv_v, b1_v,
               g0, g1, g2, g3, s0, s1, s2, s3, shared):
    c = lax.axis_index("c")
    s = lax.axis_index("s")
    pltpu.sync_copy(dinv_hbm.at[c, pl.ds(s * NR, NR)], dinv_v)
    pltpu.sync_copy(b1_hbm.at[c], b1_v)

    def _scale(rows, wbP, k):
        def _sc(j, _):
            wv = wbP[k, 0, pl.ds(j * 16, 16)]
            for l in range(16):
                ws = wv[l]
                row = j * 16 + l
                for kk in range(8):
                    sl = pl.ds(kk * 16, 16)
                    rows[row, sl] = rows[row, sl] * ws
            return 0
        lax.fori_loop(0, CH // 16, _sc, 0)

    def _sweep(gather_ref):
        def _z(i, _):
            for k in range(8):
                r1[i, pl.ds(k * 16, 16)] = jnp.zeros((16,), jnp.float32)
            return 0
        lax.fori_loop(0, CH, _z, 0)

        def _zc(t, _):
            pltpu.sync_copy(r1, shared.at[pl.ds(s * NR + t * CH, CH)])
            return 0
        lax.fori_loop(0, NR // CH, _zc, 0)
        plsc.subcore_barrier()

        pltpu.sync_copy(e_hbm.at[c, s, pl.ds(0, 2)], ebPA)
        pltpu.sync_copy(w_hbm.at[c, s, pl.ds(0, 2)], wbPA)
        pltpu.async_copy(gather_ref.at[ebPA.at[0, 0]], r0, g0)
        pltpu.async_copy(gather_ref.at[ebPA.at[1, 0]], r1, g1)

        def _quad(ii, _):
            c0 = 4 * ii

            @pl.when(ii > 0)
            def _():
                pltpu.make_async_copy(r2, shared.at[ebPB.at[0, 1]], s2).wait()
                pltpu.make_async_copy(r3, shared.at[ebPB.at[1, 1]], s3).wait()
            pltpu.sync_copy(e_hbm.at[c, s, pl.ds(c0 + 2, 2)], ebPB)
            pltpu.sync_copy(w_hbm.at[c, s, pl.ds(c0 + 2, 2)], wbPB)
            pltpu.async_copy(gather_ref.at[ebPB.at[0, 0]], r2, g2)
            pltpu.async_copy(gather_ref.at[ebPB.at[1, 0]], r3, g3)

            pltpu.make_async_copy(gather_ref.at[ebPA.at[0, 0]], r0, g0).wait()
            _scale(r0, wbPA, 0)
            pltpu.async_copy(r0, shared.at[ebPA.at[0, 1]], s0, add=True)
            pltpu.make_async_copy(gather_ref.at[ebPA.at[1, 0]], r1, g1).wait()
            _scale(r1, wbPA, 1)
            pltpu.async_copy(r1, shared.at[ebPA.at[1, 1]], s1, add=True)

            pltpu.make_async_copy(r0, shared.at[ebPA.at[0, 1]], s0).wait()
            pltpu.make_async_copy(r1, shared.at[ebPA.at[1, 1]], s1).wait()

            @pl.when(ii < NCH // 4 - 1)
            def _():
                pltpu.sync_copy(e_hbm.at[c, s, pl.ds(c0 + 4, 2)], ebPA)
                pltpu.sync_copy(w_hbm.at[c, s, pl.ds(c0 + 4, 2)], wbPA)
                pltpu.async_copy(gather_ref.at[ebPA.at[0, 0]], r0, g0)
                pltpu.async_copy(gather_ref.at[ebPA.at[1, 0]], r1, g1)

            pltpu.make_async_copy(gather_ref.at[ebPB.at[0, 0]], r2, g2).wait()
            _scale(r2, wbPB, 0)
            pltpu.async_copy(r2, shared.at[ebPB.at[0, 1]], s2, add=True)
            pltpu.make_async_copy(gather_ref.at[ebPB.at[1, 0]], r3, g3).wait()
            _scale(r3, wbPB, 1)
            pltpu.async_copy(r3, shared.at[ebPB.at[1, 1]], s3, add=True)
            return 0
        lax.fori_loop(0, NCH // 4, _quad, 0)
        pltpu.make_async_copy(r2, shared.at[ebPB.at[0, 1]], s2).wait()
        pltpu.make_async_copy(r3, shared.at[ebPB.at[1, 1]], s3).wait()

        pltpu.sync_copy(e_hbm.at[c, s, pl.ds(NCH - 1, 1)], ebPA.at[pl.ds(0, 1)])
        pltpu.sync_copy(w_hbm.at[c, s, pl.ds(NCH - 1, 1)], wbPA.at[pl.ds(0, 1)])
        pltpu.async_copy(gather_ref.at[ebPA.at[0, 0]], r0, g0)
        pltpu.make_async_copy(gather_ref.at[ebPA.at[0, 0]], r0, g0).wait()
        _scale(r0, wbPA, 0)
        pltpu.sync_copy(r0, shared.at[ebPA.at[0, 1]], add=True)
        plsc.subcore_barrier()

    def _p1(p, _):
        _sweep(h1s_hbm.at[c, p])

        def _ut(t, _):
            base = s * NR + t * ZR
            pltpu.sync_copy(shared.at[pl.ds(base, ZR)], r2.at[pl.ds(0, ZR)])
            pltpu.sync_copy(h1s_hbm.at[c, p, pl.ds(base, ZR)],
                            r3.at[pl.ds(0, ZR)])

            def _pw(g, _):
                dv = dinv_v[pl.ds(t * ZR + g * 16, 16)]
                for l in range(16):
                    dr = dv[l]
                    row = g * 16 + l
                    for k in range(8):
                        sl = pl.ds(k * 16, 16)
                        b1k = b1_v[pl.ds(p * 128 + k * 16, 16)]
                        val = (r2[row, sl] + r3[row, sl]) * dr + b1k
                        r3[row, sl] = jnp.maximum(val, 0.0) * dr
                return 0
            lax.fori_loop(0, ZR // 16, _pw, 0)
            pltpu.sync_copy(r3.at[pl.ds(0, ZR)], u_hbm.at[c, p, pl.ds(base, ZR)])
            return 0
        lax.fori_loop(0, NR // ZR, _ut, 0)
        return 0
    lax.fori_loop(0, 2, _p1, 0)
    plsc.subcore_barrier()

    def _p2(p, _):
        _sweep(u_hbm.at[c, p])

        def _vt(t, _):
            base = s * NR + t * ZR
            pltpu.sync_copy(shared.at[pl.ds(base, ZR)], r2.at[pl.ds(0, ZR)])
            pltpu.sync_copy(r2.at[pl.ds(0, ZR)], v_hbm.at[c, p, pl.ds(base, ZR)])
            return 0
        lax.fori_loop(0, NR // ZR, _vt, 0)
        return 0
    lax.fori_loop(0, 2, _p2, 0)


_conv_call = pl.kernel(
    _conv_body,
    out_type=(jax.ShapeDtypeStruct((2, 2, NP, 128), jnp.float32),
              jax.ShapeDtypeStruct((2, 2, NP, 128), jnp.float32)),
    mesh=_sc_mesh,
    scratch_types=[
        pltpu.VMEM((2, 2, CH), jnp.int32),
        pltpu.VMEM((2, 2, CH), jnp.int32),
        pltpu.VMEM((2, 1, CH), jnp.float32),
        pltpu.VMEM((2, 1, CH), jnp.float32),
        pltpu.VMEM((CH, 128), jnp.float32),
        pltpu.VMEM((CH, 128), jnp.float32),
        pltpu.VMEM((CH, 128), jnp.float32),
        pltpu.VMEM((CH, 128), jnp.float32),
        pltpu.VMEM((NR,), jnp.float32),
        pltpu.VMEM((256,), jnp.float32),
        pltpu.SemaphoreType.DMA,
        pltpu.SemaphoreType.DMA,
        pltpu.SemaphoreType.DMA,
        pltpu.SemaphoreType.DMA,
        pltpu.SemaphoreType.DMA,
        pltpu.SemaphoreType.DMA,
        pltpu.SemaphoreType.DMA,
        pltpu.SemaphoreType.DMA,
        pltpu.VMEM_SHARED((NP, 128), jnp.float32),
    ],
)


def _tc1_body(x_ref, w_ref, deg_ref, h1s_ref, dinv_ref):
    deg = deg_ref[0, :, 0] + 1.0
    dinv = jnp.where(deg > 0, lax.rsqrt(jnp.maximum(deg, 1e-12)), 0.0)
    h = jnp.dot(x_ref[0], w_ref[0], preferred_element_type=jnp.float32)
    hs = h * dinv[:, None]
    h1s_ref[0, 0] = hs[:, :128]
    h1s_ref[0, 1] = hs[:, 128:]
    dinv_ref[0] = dinv[:, None]


def _tc1(x, w1, deg):
    return pl.pallas_call(
        _tc1_body,
        grid=(2, NP // BLKP),
        in_specs=[
            pl.BlockSpec((1, BLKP, 256), lambda g, i: (g, i, 0)),
            pl.BlockSpec((1, 256, 256), lambda g, i: (g, 0, 0)),
            pl.BlockSpec((1, BLKP, 1), lambda g, i: (g, i, 0)),
        ],
        out_specs=[
            pl.BlockSpec((1, 2, BLKP, 128), lambda g, i: (g, 0, i, 0)),
            pl.BlockSpec((1, BLKP, 1), lambda g, i: (g, i, 0)),
        ],
        out_shape=[
            jax.ShapeDtypeStruct((2, 2, NP, 128), jnp.float32),
            jax.ShapeDtypeStruct((2, NP, 1), jnp.float32),
        ],
    )(x, w1, deg)


def _tc3_body(u_ref, v_ref, dinv_ref, w2_ref, b2_ref, enc_ref):
    a = jnp.concatenate(
        [u_ref[0, 0] + v_ref[0, 0], u_ref[0, 1] + v_ref[0, 1]], axis=1)
    a = a * dinv_ref[0]
    enc_ref[0] = jnp.dot(a, w2_ref[0],
                         preferred_element_type=jnp.float32) + b2_ref[0]


def _tc3(u, v, dinv, w2, b2):
    return pl.pallas_call(
        _tc3_body,
        grid=(2, N // BLK),
        in_specs=[
            pl.BlockSpec((1, 2, BLK, 128), lambda g, i: (g, 0, i, 0)),
            pl.BlockSpec((1, 2, BLK, 128), lambda g, i: (g, 0, i, 0)),
            pl.BlockSpec((1, BLK, 1), lambda g, i: (g, i, 0)),
            pl.BlockSpec((1, 256, 128), lambda g, i: (g, 0, 0)),
            pl.BlockSpec((1, 1, 128), lambda g, i: (g, 0, 0)),
        ],
        out_specs=pl.BlockSpec((1, BLK, 128), lambda g, i: (g, i, 0)),
        out_shape=jax.ShapeDtypeStruct((2, N, 128), jnp.float32),
    )(u, v, dinv, w2, b2)


def _dec_body(a_ref, b_ref, o_ref):
    z = lax.dot_general(a_ref[0], b_ref[0], (((1,), (1,)), ((), ())),
                        preferred_element_type=jnp.float32)
    o_ref[...] = 0.5 * jnp.tanh(0.5 * z) + 0.5


def _dec(enc):
    bi = 2000
    bj = 2560
    return pl.pallas_call(
        _dec_body,
        grid=(N // bi, pl.cdiv(N, bj)),
        in_specs=[
            pl.BlockSpec((1, bi, 128), lambda i, j: (0, i, 0)),
            pl.BlockSpec((1, bj, 128), lambda i, j: (1, j, 0)),
        ],
        out_specs=pl.BlockSpec((bi, bj), lambda i, j: (i, j)),
        out_shape=jax.ShapeDtypeStruct((N, N), jnp.float32),
    )(enc, enc)


def kernel(nc_x, nc_edge_index, nc_edge_weight, dis_x, dis_edge_index,
           dis_edge_weight, NW1, Nb1, NW2, Nb2, DW1, Db1, DW2, Db2):
    src = jnp.stack([nc_edge_index[0], dis_edge_index[0]]
                    ).astype(jnp.int32).reshape(2, NS, NCH, CH)
    dst = jnp.stack([nc_edge_index[1], dis_edge_index[1]]
                    ).astype(jnp.int32).reshape(2, NS, NCH, CH)
    w = jnp.stack([nc_edge_weight, dis_edge_weight]).reshape(2, NS, NCH, 1, CH)
    edges = jnp.stack([src, dst], axis=3)
    x = jnp.stack([nc_x, dis_x])
    w1 = jnp.stack([NW1, DW1])
    b1 = jnp.stack([Nb1, Db1])
    w2 = jnp.stack([NW2, DW2])
    b2 = jnp.stack([Nb2, Db2])[:, None, :]

    deg = _deg_call(edges, w)
    h1s, dinv = _tc1(x, w1, deg[:, :, None])
    u, v = _conv_call(h1s, edges, w, dinv[:, :, 0], b1)
    enc = _tc3(u, v, dinv, w2, b2)
    return _dec(enc)
